# baseline (speedup 1.0000x reference)
import jax
import jax.numpy as jnp
from jax import lax
from jax.experimental import pallas as pl
from jax.experimental.pallas import tpu as pltpu
from jax.experimental.pallas import tpu_sc as plsc

_N = 10000
_NP = 10240
_E = 320000
_D = 128
_DH = _D // 2
_G = 64
_CH = 125
_NC = 2
_NS = 16
_NW = _NC * _NS
_CPS = _E // _CH // _NS
_CPT = _E // _CH // _NW
_RPS = _NP // _NS
_ZR = 128


_KC = 2
_MC = _CPS // _KC
_NB = 2


def _sc_agg_body(x2_hbm, src_hbm, dst_hbm, part,
                 src_v, dst_v, rows, zbuf, acc, sg, ss):
  c = lax.axis_index("c")
  s = lax.axis_index("s")

  def zrow(i, _):
    for k in range(_DH // 16):
      zbuf[i, pl.ds(k * 16, 16)] = jnp.zeros((16,), jnp.float32)
    return 0
  lax.fori_loop(0, _ZR, zrow, 0)
  for k in range(_RPS // _ZR):
    pltpu.sync_copy(zbuf, acc.at[pl.ds(s * _RPS + k * _ZR, _ZR)])

  pltpu.sync_copy(src_hbm.at[s], src_v)
  pltpu.sync_copy(dst_hbm.at[s], dst_v)
  plsc.subcore_barrier()

  for b in range(_NB):
    pltpu.async_copy(x2_hbm.at[c].at[src_v.at[b]], rows[b], sg[b])

  def step(i, _):
    j0 = i * _NB
    for b in range(_NB):
      j = j0 + b
      pltpu.make_async_copy(x2_hbm.at[c].at[src_v.at[j]], rows[b],
                            sg[b]).wait()
      pltpu.async_copy(rows[b], acc.at[dst_v.at[j]], ss[b], add=True)
    for b in range(_NB):
      jn = jnp.minimum(j0 + b + _NB, _MC - 1)
      pltpu.make_async_copy(rows[b], acc.at[dst_v.at[j0 + b]],
                            ss[b]).wait()
      pltpu.async_copy(x2_hbm.at[c].at[src_v.at[jn]], rows[b], sg[b])
    return 0
  lax.fori_loop(0, _MC // _NB, step, 0)
  for b in range(_NB):
    pltpu.make_async_copy(x2_hbm.at[c].at[src_v.at[0]], rows[b],
                          sg[b]).wait()

  plsc.subcore_barrier()
  pltpu.sync_copy(acc.at[pl.ds(s * _RPS, _RPS)],
                  part.at[c].at[pl.ds(s * _RPS, _RPS)])


def _sc_deg_body(dst_hbm, partdeg, dst_v, ones_v, zdeg, accd):
  c = lax.axis_index("c")
  s = lax.axis_index("s")
  wid = s * _NC + c

  def zdrow(i, _):
    zdeg[i, :] = jnp.zeros((16,), jnp.float32)
    return 0
  lax.fori_loop(0, _RPS, zdrow, 0)
  pltpu.sync_copy(zdeg, accd.at[pl.ds(s * _RPS, _RPS)])

  def orow(i, _):
    ones_v[i, :] = jnp.ones((16,), jnp.float32)
    return 0
  lax.fori_loop(0, _CH, orow, 0)

  pltpu.sync_copy(dst_hbm.at[wid], dst_v)
  plsc.subcore_barrier()

  def step(j, _):
    pltpu.sync_copy(ones_v, accd.at[dst_v.at[j]], add=True)
    return 0
  lax.fori_loop(0, _CPT, step, 0)

  plsc.subcore_barrier()
  pltpu.sync_copy(accd.at[pl.ds(s * _RPS, _RPS)],
                  partdeg.at[c].at[pl.ds(s * _RPS, _RPS)])


def _sc_mesh():
  return plsc.VectorSubcoreMesh(core_axis_name="c", subcore_axis_name="s")


_agg = pl.kernel(
    _sc_agg_body,
    out_type=jax.ShapeDtypeStruct((_NC, _NP, _DH), jnp.float32),
    mesh=_sc_mesh(),
    compiler_params=pltpu.CompilerParams(use_tc_tiling_on_sc=False),
    scratch_types=[
        pltpu.VMEM((_MC, _KC * _CH), jnp.int32),
        pltpu.VMEM((_MC, _KC * _CH), jnp.int32),
        [pltpu.VMEM((_KC * _CH, _DH), jnp.float32) for _ in range(_NB)],
        pltpu.VMEM((_ZR, _DH), jnp.float32),
        pltpu.VMEM_SHARED((_NP, _DH), jnp.float32),
        [pltpu.SemaphoreType.DMA for _ in range(_NB)],
        [pltpu.SemaphoreType.DMA for _ in range(_NB)],
    ],
)

_deg = pl.kernel(
    _sc_deg_body,
    out_type=jax.ShapeDtypeStruct((_NC, _NP, 16), jnp.float32),
    mesh=_sc_mesh(),
    compiler_params=pltpu.CompilerParams(use_tc_tiling_on_sc=False),
    scratch_types=[
        pltpu.VMEM((_CPT, _CH), jnp.int32),
        pltpu.VMEM((_CH, 16), jnp.float32),
        pltpu.VMEM((_RPS, 16), jnp.float32),
        pltpu.VMEM_SHARED((_NP, 16), jnp.float32),
    ],
)

_BN = 2048


def _tc_layer1_body(p2, d0, d1, W, b, h2, scale, maskb):
  deg = (d0[...] + d1[...])[:, 0:1]
  sc = 1.0 / jnp.maximum(deg, 1.0)
  mk = (deg > 0.0).astype(jnp.float32)
  mean = jnp.concatenate([p2[0], p2[1]], axis=1) * sc
  z = lax.dot_general(mean, W[...], (((1,), (1,)), ((), ())),
                      preferred_element_type=jnp.float32)
  z = z + b[...] * mk
  ht = jnp.tanh(jnp.maximum(z, 0.0))
  h2[0] = ht[:, :_DH]
  h2[1] = ht[:, _DH:]
  scale[...] = sc
  maskb[...] = mk


def _tc_layerk_body(p2, sc_ref, mk_ref, W, b, h2):
  mean = jnp.concatenate([p2[0], p2[1]], axis=1) * sc_ref[...]
  z = lax.dot_general(mean, W[...], (((1,), (1,)), ((), ())),
                      preferred_element_type=jnp.float32)
  z = z + b[...] * mk_ref[...]
  ht = jnp.tanh(jnp.maximum(z, 0.0))
  h2[0] = ht[:, :_DH]
  h2[1] = ht[:, _DH:]


def _blk2():
  return pl.BlockSpec((2, _BN, _DH), lambda i: (0, i, 0))


def _blk(r, c):
  return pl.BlockSpec((r, c), lambda i: (i, 0))


def _fix(r, c):
  return pl.BlockSpec((r, c), lambda i: (0, 0))


def _tc_layer1(p2, d0, d1, W, b):
  return pl.pallas_call(
      _tc_layer1_body,
      grid=(_NP // _BN,),
      in_specs=[_blk2(), _blk(_BN, 16), _blk(_BN, 16),
                _fix(_D, _D), _fix(1, _D)],
      out_specs=[_blk2(), _blk(_BN, 1), _blk(_BN, 1)],
      out_shape=[jax.ShapeDtypeStruct((2, _NP, _DH), jnp.float32),
                 jax.ShapeDtypeStruct((_NP, 1), jnp.float32),
                 jax.ShapeDtypeStruct((_NP, 1), jnp.float32)],
  )(p2, d0, d1, W, b)


def _tc_layerk(p2, scale, maskb, W, b):
  return pl.pallas_call(
      _tc_layerk_body,
      grid=(_NP // _BN,),
      in_specs=[_blk2(), _blk(_BN, 1), _blk(_BN, 1),
                _fix(_D, _D), _fix(1, _D)],
      out_specs=_blk2(),
      out_shape=jax.ShapeDtypeStruct((2, _NP, _DH), jnp.float32),
  )(p2, scale, maskb, W, b)


def _tc_pool_body(h2, bt, Wfc, bfc, Wr, br, out, accs, cnts):
  i = pl.program_id(0)

  @pl.when(i == 0)
  def _():
    accs[...] = jnp.zeros_like(accs)
    cnts[...] = jnp.zeros_like(cnts)

  g = lax.broadcasted_iota(jnp.int32, (1, _G), 1)
  M = (bt[...] == g).astype(jnp.float32)
  h4 = jnp.concatenate([h2[0], h2[1]], axis=1)
  accs[...] += lax.dot_general(M, h4, (((0,), (0,)), ((), ())),
                               preferred_element_type=jnp.float32)
  cnts[...] += jnp.sum(M, axis=0)[:, None]

  @pl.when(i == pl.num_programs(0) - 1)
  def _():
    pooled = accs[...] / jnp.maximum(cnts[...], 1.0)
    hfc = lax.dot_general(pooled, Wfc[...], (((1,), (1,)), ((), ())),
                          preferred_element_type=jnp.float32) + bfc[...]
    hfc = jnp.maximum(hfc, 0.0)
    out[...] = lax.dot_general(hfc, Wr[...], (((1,), (1,)), ((), ())),
                               preferred_element_type=jnp.float32) + br[...]


def _tc_pool(h2, bt, Wfc, bfc, Wr, br):
  return pl.pallas_call(
      _tc_pool_body,
      grid=(_NP // _BN,),
      in_specs=[_blk2(), _blk(_BN, 1), _fix(_D, _D), _fix(1, _D),
                _fix(8, _D), _fix(1, 8)],
      out_specs=_fix(_G, 8),
      out_shape=jax.ShapeDtypeStruct((_G, 8), jnp.float32),
      scratch_shapes=[pltpu.VMEM((_G, _D), jnp.float32),
                      pltpu.VMEM((_G, _D), jnp.float32)],
  )(h2, bt, Wfc, bfc, Wr, br)


def kernel(x, edge_index, batch, W1, b1, W2, b2, W3, b3, W4, b4,
           Wfc, bfc, Wr, br):
  src16 = edge_index[0].reshape(_NS, _MC, _KC * _CH)
  dst16 = edge_index[1].reshape(_NS, _MC, _KC * _CH)
  dst32 = edge_index[1].reshape(_NW, _CPT, _CH)

  xp = jnp.concatenate(
      [x, jnp.zeros((_NP - _N, _D), jnp.float32)]).reshape(_NP, 2, _DH)
  x2 = jnp.transpose(xp, (1, 0, 2))

  pdeg = _deg(dst32)
  p2 = _agg(x2, src16, dst16)
  h2, scale, maskb = _tc_layer1(p2, pdeg[0], pdeg[1], W1, b1.reshape(1, _D))
  for W, b in ((W2, b2), (W3, b3), (W4, b4)):
    p2 = _agg(h2, src16, dst16)
    h2 = _tc_layerk(p2, scale, maskb, W, b.reshape(1, _D))

  batch_p = jnp.concatenate(
      [batch, jnp.full((_NP - _N,), _G, jnp.int32)]).reshape(_NP, 1)
  out = _tc_pool(h2, batch_p, Wfc, bfc.reshape(1, _D),
                 Wr, br.reshape(1, 8))
  return out


# speedup vs baseline: 1.1664x; 1.1664x over previous
import jax
import jax.numpy as jnp
from jax import lax
from jax.experimental import pallas as pl
from jax.experimental.pallas import tpu as pltpu
from jax.experimental.pallas import tpu_sc as plsc

_N = 10000
_NP = 10240
_E = 320000
_D = 128
_DH = _D // 2
_G = 64
_CH = 125
_NC = 2
_NS = 16
_NW = _NC * _NS
_CPS = _E // _CH // _NS
_CPT = _E // _CH // _NW
_RPS = _NP // _NS
_ZR = 128


_KC = 2
_MC = _CPS // _KC
_NPH = 2
_MCP = _MC // _NPH
_NB = 4


def _sc_agg_body(x2_hbm, src_hbm, dst_hbm, part,
                 src_v, dst_v, rows, acc, sg, ss):
  c = lax.axis_index("c")
  s = lax.axis_index("s")

  def zrow(i, _):
    for k in range(_DH // 16):
      rows[0][i, pl.ds(k * 16, 16)] = jnp.zeros((16,), jnp.float32)
    return 0
  lax.fori_loop(0, _ZR, zrow, 0)
  for k in range(_RPS // _ZR):
    pltpu.sync_copy(rows[0].at[pl.ds(0, _ZR)],
                    acc.at[pl.ds(s * _RPS + k * _ZR, _ZR)])

  plsc.subcore_barrier()

  for p in range(_NPH):
    pltpu.sync_copy(src_hbm.at[s].at[pl.ds(p * _MCP, _MCP)], src_v)
    pltpu.sync_copy(dst_hbm.at[s].at[pl.ds(p * _MCP, _MCP)], dst_v)
    for b in range(_NB):
      pltpu.async_copy(x2_hbm.at[c].at[src_v.at[b]], rows[b], sg[b])

    def step(i, _):
      j0 = i * _NB
      for b in range(_NB):
        j = j0 + b
        pltpu.make_async_copy(x2_hbm.at[c].at[src_v.at[j]], rows[b],
                              sg[b]).wait()
        pltpu.async_copy(rows[b], acc.at[dst_v.at[j]], ss[b], add=True)
      for b in range(_NB):
        jn = jnp.minimum(j0 + b + _NB, _MCP - 1)
        pltpu.make_async_copy(rows[b], acc.at[dst_v.at[j0 + b]],
                              ss[b]).wait()
        pltpu.async_copy(x2_hbm.at[c].at[src_v.at[jn]], rows[b], sg[b])
      return 0
    lax.fori_loop(0, _MCP // _NB, step, 0)
    for b in range(_NB):
      pltpu.make_async_copy(x2_hbm.at[c].at[src_v.at[0]], rows[b],
                            sg[b]).wait()

  plsc.subcore_barrier()
  pltpu.sync_copy(acc.at[pl.ds(s * _RPS, _RPS)],
                  part.at[c].at[pl.ds(s * _RPS, _RPS)])


def _sc_deg_body(dst_hbm, partdeg, dst_v, ones_v, zdeg, accd):
  c = lax.axis_index("c")
  s = lax.axis_index("s")
  wid = s * _NC + c

  def zdrow(i, _):
    zdeg[i, :] = jnp.zeros((16,), jnp.float32)
    return 0
  lax.fori_loop(0, _RPS, zdrow, 0)
  pltpu.sync_copy(zdeg, accd.at[pl.ds(s * _RPS, _RPS)])

  def orow(i, _):
    ones_v[i, :] = jnp.ones((16,), jnp.float32)
    return 0
  lax.fori_loop(0, _CH, orow, 0)

  pltpu.sync_copy(dst_hbm.at[wid], dst_v)
  plsc.subcore_barrier()

  def step(j, _):
    pltpu.sync_copy(ones_v, accd.at[dst_v.at[j]], add=True)
    return 0
  lax.fori_loop(0, _CPT, step, 0)

  plsc.subcore_barrier()
  pltpu.sync_copy(accd.at[pl.ds(s * _RPS, _RPS)],
                  partdeg.at[c].at[pl.ds(s * _RPS, _RPS)])


def _sc_mesh():
  return plsc.VectorSubcoreMesh(core_axis_name="c", subcore_axis_name="s")


_agg = pl.kernel(
    _sc_agg_body,
    out_type=jax.ShapeDtypeStruct((_NC, _NP, _DH), jnp.float32),
    mesh=_sc_mesh(),
    compiler_params=pltpu.CompilerParams(use_tc_tiling_on_sc=False),
    scratch_types=[
        pltpu.VMEM((_MCP, _KC * _CH), jnp.int32),
        pltpu.VMEM((_MCP, _KC * _CH), jnp.int32),
        [pltpu.VMEM((_KC * _CH, _DH), jnp.float32) for _ in range(_NB)],
        pltpu.VMEM_SHARED((_NP, _DH), jnp.float32),
        [pltpu.SemaphoreType.DMA for _ in range(_NB)],
        [pltpu.SemaphoreType.DMA for _ in range(_NB)],
    ],
)

_deg = pl.kernel(
    _sc_deg_body,
    out_type=jax.ShapeDtypeStruct((_NC, _NP, 16), jnp.float32),
    mesh=_sc_mesh(),
    compiler_params=pltpu.CompilerParams(use_tc_tiling_on_sc=False),
    scratch_types=[
        pltpu.VMEM((_CPT, _CH), jnp.int32),
        pltpu.VMEM((_CH, 16), jnp.float32),
        pltpu.VMEM((_RPS, 16), jnp.float32),
        pltpu.VMEM_SHARED((_NP, 16), jnp.float32),
    ],
)

_BN = 2048


def _tc_layer1_body(p2, d0, d1, W, b, h2, scale, maskb):
  deg = (d0[...] + d1[...])[:, 0:1]
  sc = 1.0 / jnp.maximum(deg, 1.0)
  mk = (deg > 0.0).astype(jnp.float32)
  mean = jnp.concatenate([p2[0], p2[1]], axis=1) * sc
  z = lax.dot_general(mean, W[...], (((1,), (1,)), ((), ())),
                      preferred_element_type=jnp.float32)
  z = z + b[...] * mk
  ht = jnp.tanh(jnp.maximum(z, 0.0))
  h2[0] = ht[:, :_DH]
  h2[1] = ht[:, _DH:]
  scale[...] = sc
  maskb[...] = mk


def _tc_layerk_body(p2, sc_ref, mk_ref, W, b, h2):
  mean = jnp.concatenate([p2[0], p2[1]], axis=1) * sc_ref[...]
  z = lax.dot_general(mean, W[...], (((1,), (1,)), ((), ())),
                      preferred_element_type=jnp.float32)
  z = z + b[...] * mk_ref[...]
  ht = jnp.tanh(jnp.maximum(z, 0.0))
  h2[0] = ht[:, :_DH]
  h2[1] = ht[:, _DH:]


def _blk2():
  return pl.BlockSpec((2, _BN, _DH), lambda i: (0, i, 0))


def _blk(r, c):
  return pl.BlockSpec((r, c), lambda i: (i, 0))


def _fix(r, c):
  return pl.BlockSpec((r, c), lambda i: (0, 0))


def _tc_layer1(p2, d0, d1, W, b):
  return pl.pallas_call(
      _tc_layer1_body,
      grid=(_NP // _BN,),
      in_specs=[_blk2(), _blk(_BN, 16), _blk(_BN, 16),
                _fix(_D, _D), _fix(1, _D)],
      out_specs=[_blk2(), _blk(_BN, 1), _blk(_BN, 1)],
      out_shape=[jax.ShapeDtypeStruct((2, _NP, _DH), jnp.float32),
                 jax.ShapeDtypeStruct((_NP, 1), jnp.float32),
                 jax.ShapeDtypeStruct((_NP, 1), jnp.float32)],
  )(p2, d0, d1, W, b)


def _tc_layerk(p2, scale, maskb, W, b):
  return pl.pallas_call(
      _tc_layerk_body,
      grid=(_NP // _BN,),
      in_specs=[_blk2(), _blk(_BN, 1), _blk(_BN, 1),
                _fix(_D, _D), _fix(1, _D)],
      out_specs=_blk2(),
      out_shape=jax.ShapeDtypeStruct((2, _NP, _DH), jnp.float32),
  )(p2, scale, maskb, W, b)


def _tc_pool_body(h2, bt, Wfc, bfc, Wr, br, out, accs, cnts):
  i = pl.program_id(0)

  @pl.when(i == 0)
  def _():
    accs[...] = jnp.zeros_like(accs)
    cnts[...] = jnp.zeros_like(cnts)

  g = lax.broadcasted_iota(jnp.int32, (1, _G), 1)
  M = (bt[...] == g).astype(jnp.float32)
  h4 = jnp.concatenate([h2[0], h2[1]], axis=1)
  accs[...] += lax.dot_general(M, h4, (((0,), (0,)), ((), ())),
                               preferred_element_type=jnp.float32)
  cnts[...] += jnp.sum(M, axis=0)[:, None]

  @pl.when(i == pl.num_programs(0) - 1)
  def _():
    pooled = accs[...] / jnp.maximum(cnts[...], 1.0)
    hfc = lax.dot_general(pooled, Wfc[...], (((1,), (1,)), ((), ())),
                          preferred_element_type=jnp.float32) + bfc[...]
    hfc = jnp.maximum(hfc, 0.0)
    out[...] = lax.dot_general(hfc, Wr[...], (((1,), (1,)), ((), ())),
                               preferred_element_type=jnp.float32) + br[...]


def _tc_pool(h2, bt, Wfc, bfc, Wr, br):
  return pl.pallas_call(
      _tc_pool_body,
      grid=(_NP // _BN,),
      in_specs=[_blk2(), _blk(_BN, 1), _fix(_D, _D), _fix(1, _D),
                _fix(8, _D), _fix(1, 8)],
      out_specs=_fix(_G, 8),
      out_shape=jax.ShapeDtypeStruct((_G, 8), jnp.float32),
      scratch_shapes=[pltpu.VMEM((_G, _D), jnp.float32),
                      pltpu.VMEM((_G, _D), jnp.float32)],
  )(h2, bt, Wfc, bfc, Wr, br)


def kernel(x, edge_index, batch, W1, b1, W2, b2, W3, b3, W4, b4,
           Wfc, bfc, Wr, br):
  src16 = edge_index[0].reshape(_NS, _MC, _KC * _CH)
  dst16 = edge_index[1].reshape(_NS, _MC, _KC * _CH)
  dst32 = edge_index[1].reshape(_NW, _CPT, _CH)

  xp = jnp.concatenate(
      [x, jnp.zeros((_NP - _N, _D), jnp.float32)]).reshape(_NP, 2, _DH)
  x2 = jnp.transpose(xp, (1, 0, 2))

  pdeg = _deg(dst32)
  p2 = _agg(x2, src16, dst16)
  h2, scale, maskb = _tc_layer1(p2, pdeg[0], pdeg[1], W1, b1.reshape(1, _D))
  for W, b in ((W2, b2), (W3, b3), (W4, b4)):
    p2 = _agg(h2, src16, dst16)
    h2 = _tc_layerk(p2, scale, maskb, W, b.reshape(1, _D))

  batch_p = jnp.concatenate(
      [batch, jnp.full((_NP - _N,), _G, jnp.int32)]).reshape(_NP, 1)
  out = _tc_pool(h2, batch_p, Wfc, bfc.reshape(1, _D),
                 Wr, br.reshape(1, 8))
  return out


# speedup vs baseline: 1.1960x; 1.0253x over previous
import jax
import jax.numpy as jnp
from jax import lax
from jax.experimental import pallas as pl
from jax.experimental.pallas import tpu as pltpu
from jax.experimental.pallas import tpu_sc as plsc

_N = 10000
_NP = 10240
_E = 320000
_D = 128
_DH = _D // 2
_G = 64
_CH = 125
_NC = 2
_NS = 16
_NW = _NC * _NS
_CPS = _E // _CH // _NS
_CPT = _E // _CH // _NW
_RPS = _NP // _NS
_ZR = 128


_KC = 1
_MC = _CPS // _KC
_NPH = 2
_MCP = _MC // _NPH
_NB = 8


def _sc_agg_body(x2_hbm, src_hbm, dst_hbm, part,
                 src_v, dst_v, rows, acc, sg, ss):
  c = lax.axis_index("c")
  s = lax.axis_index("s")

  def zrow(i, _):
    for k in range(_DH // 16):
      rows[0][i, pl.ds(k * 16, 16)] = jnp.zeros((16,), jnp.float32)
    return 0
  lax.fori_loop(0, _ZR, zrow, 0)
  for k in range(_RPS // _ZR):
    pltpu.sync_copy(rows[0].at[pl.ds(0, _ZR)],
                    acc.at[pl.ds(s * _RPS + k * _ZR, _ZR)])

  plsc.subcore_barrier()

  for p in range(_NPH):
    pltpu.sync_copy(src_hbm.at[s].at[pl.ds(p * _MCP, _MCP)], src_v)
    pltpu.sync_copy(dst_hbm.at[s].at[pl.ds(p * _MCP, _MCP)], dst_v)
    for b in range(_NB):
      pltpu.async_copy(x2_hbm.at[c].at[src_v.at[b]], rows[b], sg[b])

    def step(i, _):
      j0 = i * _NB
      for b in range(_NB):
        j = j0 + b
        pltpu.make_async_copy(x2_hbm.at[c].at[src_v.at[j]], rows[b],
                              sg[b]).wait()
        pltpu.async_copy(rows[b], acc.at[dst_v.at[j]], ss[b], add=True)
      for b in range(_NB):
        jn = jnp.minimum(j0 + b + _NB, _MCP - 1)
        pltpu.make_async_copy(rows[b], acc.at[dst_v.at[j0 + b]],
                              ss[b]).wait()
        pltpu.async_copy(x2_hbm.at[c].at[src_v.at[jn]], rows[b], sg[b])
      return 0
    lax.fori_loop(0, _MCP // _NB, step, 0)
    for b in range(_NB):
      pltpu.make_async_copy(x2_hbm.at[c].at[src_v.at[0]], rows[b],
                            sg[b]).wait()

  plsc.subcore_barrier()
  pltpu.sync_copy(acc.at[pl.ds(s * _RPS, _RPS)],
                  part.at[c].at[pl.ds(s * _RPS, _RPS)])


def _sc_deg_body(dst_hbm, partdeg, dst_v, ones_v, zdeg, accd):
  c = lax.axis_index("c")
  s = lax.axis_index("s")
  wid = s * _NC + c

  def zdrow(i, _):
    zdeg[i, :] = jnp.zeros((16,), jnp.float32)
    return 0
  lax.fori_loop(0, _RPS, zdrow, 0)
  pltpu.sync_copy(zdeg, accd.at[pl.ds(s * _RPS, _RPS)])

  def orow(i, _):
    ones_v[i, :] = jnp.ones((16,), jnp.float32)
    return 0
  lax.fori_loop(0, _CH, orow, 0)

  pltpu.sync_copy(dst_hbm.at[wid], dst_v)
  plsc.subcore_barrier()

  def step(j, _):
    pltpu.sync_copy(ones_v, accd.at[dst_v.at[j]], add=True)
    return 0
  lax.fori_loop(0, _CPT, step, 0)

  plsc.subcore_barrier()
  pltpu.sync_copy(accd.at[pl.ds(s * _RPS, _RPS)],
                  partdeg.at[c].at[pl.ds(s * _RPS, _RPS)])


def _sc_mesh():
  return plsc.VectorSubcoreMesh(core_axis_name="c", subcore_axis_name="s")


_agg = pl.kernel(
    _sc_agg_body,
    out_type=jax.ShapeDtypeStruct((_NC, _NP, _DH), jnp.float32),
    mesh=_sc_mesh(),
    compiler_params=pltpu.CompilerParams(use_tc_tiling_on_sc=False),
    scratch_types=[
        pltpu.VMEM((_MCP, _KC * _CH), jnp.int32),
        pltpu.VMEM((_MCP, _KC * _CH), jnp.int32),
        [pltpu.VMEM((_KC * _CH, _DH), jnp.float32) for _ in range(_NB)],
        pltpu.VMEM_SHARED((_NP, _DH), jnp.float32),
        [pltpu.SemaphoreType.DMA for _ in range(_NB)],
        [pltpu.SemaphoreType.DMA for _ in range(_NB)],
    ],
)

_deg = pl.kernel(
    _sc_deg_body,
    out_type=jax.ShapeDtypeStruct((_NC, _NP, 16), jnp.float32),
    mesh=_sc_mesh(),
    compiler_params=pltpu.CompilerParams(use_tc_tiling_on_sc=False),
    scratch_types=[
        pltpu.VMEM((_CPT, _CH), jnp.int32),
        pltpu.VMEM((_CH, 16), jnp.float32),
        pltpu.VMEM((_RPS, 16), jnp.float32),
        pltpu.VMEM_SHARED((_NP, 16), jnp.float32),
    ],
)

_BN = 2048


def _tc_layer1_body(p2, d0, d1, W, b, h2, scale, maskb):
  deg = (d0[...] + d1[...])[:, 0:1]
  sc = 1.0 / jnp.maximum(deg, 1.0)
  mk = (deg > 0.0).astype(jnp.float32)
  mean = jnp.concatenate([p2[0], p2[1]], axis=1) * sc
  z = lax.dot_general(mean, W[...], (((1,), (1,)), ((), ())),
                      preferred_element_type=jnp.float32)
  z = z + b[...] * mk
  ht = jnp.tanh(jnp.maximum(z, 0.0))
  h2[0] = ht[:, :_DH]
  h2[1] = ht[:, _DH:]
  scale[...] = sc
  maskb[...] = mk


def _tc_layerk_body(p2, sc_ref, mk_ref, W, b, h2):
  mean = jnp.concatenate([p2[0], p2[1]], axis=1) * sc_ref[...]
  z = lax.dot_general(mean, W[...], (((1,), (1,)), ((), ())),
                      preferred_element_type=jnp.float32)
  z = z + b[...] * mk_ref[...]
  ht = jnp.tanh(jnp.maximum(z, 0.0))
  h2[0] = ht[:, :_DH]
  h2[1] = ht[:, _DH:]


def _blk2():
  return pl.BlockSpec((2, _BN, _DH), lambda i: (0, i, 0))


def _blk(r, c):
  return pl.BlockSpec((r, c), lambda i: (i, 0))


def _fix(r, c):
  return pl.BlockSpec((r, c), lambda i: (0, 0))


def _tc_layer1(p2, d0, d1, W, b):
  return pl.pallas_call(
      _tc_layer1_body,
      grid=(_NP // _BN,),
      in_specs=[_blk2(), _blk(_BN, 16), _blk(_BN, 16),
                _fix(_D, _D), _fix(1, _D)],
      out_specs=[_blk2(), _blk(_BN, 1), _blk(_BN, 1)],
      out_shape=[jax.ShapeDtypeStruct((2, _NP, _DH), jnp.float32),
                 jax.ShapeDtypeStruct((_NP, 1), jnp.float32),
                 jax.ShapeDtypeStruct((_NP, 1), jnp.float32)],
  )(p2, d0, d1, W, b)


def _tc_layerk(p2, scale, maskb, W, b):
  return pl.pallas_call(
      _tc_layerk_body,
      grid=(_NP // _BN,),
      in_specs=[_blk2(), _blk(_BN, 1), _blk(_BN, 1),
                _fix(_D, _D), _fix(1, _D)],
      out_specs=_blk2(),
      out_shape=jax.ShapeDtypeStruct((2, _NP, _DH), jnp.float32),
  )(p2, scale, maskb, W, b)


def _tc_pool_body(h2, bt, Wfc, bfc, Wr, br, out, accs, cnts):
  i = pl.program_id(0)

  @pl.when(i == 0)
  def _():
    accs[...] = jnp.zeros_like(accs)
    cnts[...] = jnp.zeros_like(cnts)

  g = lax.broadcasted_iota(jnp.int32, (1, _G), 1)
  M = (bt[...] == g).astype(jnp.float32)
  h4 = jnp.concatenate([h2[0], h2[1]], axis=1)
  accs[...] += lax.dot_general(M, h4, (((0,), (0,)), ((), ())),
                               preferred_element_type=jnp.float32)
  cnts[...] += jnp.sum(M, axis=0)[:, None]

  @pl.when(i == pl.num_programs(0) - 1)
  def _():
    pooled = accs[...] / jnp.maximum(cnts[...], 1.0)
    hfc = lax.dot_general(pooled, Wfc[...], (((1,), (1,)), ((), ())),
                          preferred_element_type=jnp.float32) + bfc[...]
    hfc = jnp.maximum(hfc, 0.0)
    out[...] = lax.dot_general(hfc, Wr[...], (((1,), (1,)), ((), ())),
                               preferred_element_type=jnp.float32) + br[...]


def _tc_pool(h2, bt, Wfc, bfc, Wr, br):
  return pl.pallas_call(
      _tc_pool_body,
      grid=(_NP // _BN,),
      in_specs=[_blk2(), _blk(_BN, 1), _fix(_D, _D), _fix(1, _D),
                _fix(8, _D), _fix(1, 8)],
      out_specs=_fix(_G, 8),
      out_shape=jax.ShapeDtypeStruct((_G, 8), jnp.float32),
      scratch_shapes=[pltpu.VMEM((_G, _D), jnp.float32),
                      pltpu.VMEM((_G, _D), jnp.float32)],
  )(h2, bt, Wfc, bfc, Wr, br)


def kernel(x, edge_index, batch, W1, b1, W2, b2, W3, b3, W4, b4,
           Wfc, bfc, Wr, br):
  src16 = edge_index[0].reshape(_NS, _MC, _KC * _CH)
  dst16 = edge_index[1].reshape(_NS, _MC, _KC * _CH)
  dst32 = edge_index[1].reshape(_NW, _CPT, _CH)

  xp = jnp.concatenate(
      [x, jnp.zeros((_NP - _N, _D), jnp.float32)]).reshape(_NP, 2, _DH)
  x2 = jnp.transpose(xp, (1, 0, 2))

  pdeg = _deg(dst32)
  p2 = _agg(x2, src16, dst16)
  h2, scale, maskb = _tc_layer1(p2, pdeg[0], pdeg[1], W1, b1.reshape(1, _D))
  for W, b in ((W2, b2), (W3, b3), (W4, b4)):
    p2 = _agg(h2, src16, dst16)
    h2 = _tc_layerk(p2, scale, maskb, W, b.reshape(1, _D))

  batch_p = jnp.concatenate(
      [batch, jnp.full((_NP - _N,), _G, jnp.int32)]).reshape(_NP, 1)
  out = _tc_pool(h2, batch_p, Wfc, bfc.reshape(1, _D),
                 Wr, br.reshape(1, 8))
  return out


# speedup vs baseline: 1.2293x; 1.0279x over previous
import jax
import jax.numpy as jnp
from jax import lax
from jax.experimental import pallas as pl
from jax.experimental.pallas import tpu as pltpu
from jax.experimental.pallas import tpu_sc as plsc

_N = 10000
_NP = 10240
_E = 320000
_D = 128
_DH = _D // 2
_G = 64
_CH = 125
_NC = 2
_NS = 16
_NW = _NC * _NS
_CPS = _E // _CH // _NS
_CPT = _E // _CH // _NW
_RPS = _NP // _NS
_ZR = 128


_KC = 1
_MC = _CPS // _KC
_NPH = 1
_MCP = _MC // _NPH
_NB = 5


def _sc_agg_body(x2_hbm, src_hbm, dst_hbm, part,
                 src_v, dst_v, rows, acc, sg, ss):
  c = lax.axis_index("c")
  s = lax.axis_index("s")

  def zrow(i, _):
    for k in range(_DH // 16):
      rows[0][i, pl.ds(k * 16, 16)] = jnp.zeros((16,), jnp.float32)
    return 0
  lax.fori_loop(0, _ZR, zrow, 0)
  for k in range(_RPS // _ZR):
    pltpu.sync_copy(rows[0].at[pl.ds(0, _ZR)],
                    acc.at[pl.ds(s * _RPS + k * _ZR, _ZR)])

  plsc.subcore_barrier()

  for p in range(_NPH):
    pltpu.sync_copy(src_hbm.at[s].at[pl.ds(p * _MCP, _MCP)], src_v)
    pltpu.sync_copy(dst_hbm.at[s].at[pl.ds(p * _MCP, _MCP)], dst_v)
    for b in range(_NB):
      pltpu.async_copy(x2_hbm.at[c].at[src_v.at[b]], rows[b], sg[b])

    def step(i, _):
      j0 = i * _NB
      for b in range(_NB):
        j = j0 + b
        pltpu.make_async_copy(x2_hbm.at[c].at[src_v.at[j]], rows[b],
                              sg[b]).wait()
        pltpu.async_copy(rows[b], acc.at[dst_v.at[j]], ss[b], add=True)
      for b in range(_NB):
        jn = jnp.minimum(j0 + b + _NB, _MCP - 1)
        pltpu.make_async_copy(rows[b], acc.at[dst_v.at[j0 + b]],
                              ss[b]).wait()
        pltpu.async_copy(x2_hbm.at[c].at[src_v.at[jn]], rows[b], sg[b])
      return 0
    lax.fori_loop(0, _MCP // _NB, step, 0)
    for b in range(_NB):
      pltpu.make_async_copy(x2_hbm.at[c].at[src_v.at[0]], rows[b],
                            sg[b]).wait()

  plsc.subcore_barrier()
  pltpu.sync_copy(acc.at[pl.ds(s * _RPS, _RPS)],
                  part.at[c].at[pl.ds(s * _RPS, _RPS)])


def _sc_deg_body(dst_hbm, partdeg, dst_v, ones_v, zdeg, accd):
  c = lax.axis_index("c")
  s = lax.axis_index("s")
  wid = s * _NC + c

  def zdrow(i, _):
    zdeg[i, :] = jnp.zeros((16,), jnp.float32)
    return 0
  lax.fori_loop(0, _RPS, zdrow, 0)
  pltpu.sync_copy(zdeg, accd.at[pl.ds(s * _RPS, _RPS)])

  def orow(i, _):
    ones_v[i, :] = jnp.ones((16,), jnp.float32)
    return 0
  lax.fori_loop(0, _CH, orow, 0)

  pltpu.sync_copy(dst_hbm.at[wid], dst_v)
  plsc.subcore_barrier()

  def step(j, _):
    pltpu.sync_copy(ones_v, accd.at[dst_v.at[j]], add=True)
    return 0
  lax.fori_loop(0, _CPT, step, 0)

  plsc.subcore_barrier()
  pltpu.sync_copy(accd.at[pl.ds(s * _RPS, _RPS)],
                  partdeg.at[c].at[pl.ds(s * _RPS, _RPS)])


def _sc_mesh():
  return plsc.VectorSubcoreMesh(core_axis_name="c", subcore_axis_name="s")


_agg = pl.kernel(
    _sc_agg_body,
    out_type=jax.ShapeDtypeStruct((_NC, _NP, _DH), jnp.float32),
    mesh=_sc_mesh(),
    compiler_params=pltpu.CompilerParams(use_tc_tiling_on_sc=False),
    scratch_types=[
        pltpu.VMEM((_MCP, _KC * _CH), jnp.int32),
        pltpu.VMEM((_MCP, _KC * _CH), jnp.int32),
        [pltpu.VMEM((_KC * _CH, _DH), jnp.float32) for _ in range(_NB)],
        pltpu.VMEM_SHARED((_NP, _DH), jnp.float32),
        [pltpu.SemaphoreType.DMA for _ in range(_NB)],
        [pltpu.SemaphoreType.DMA for _ in range(_NB)],
    ],
)

_deg = pl.kernel(
    _sc_deg_body,
    out_type=jax.ShapeDtypeStruct((_NC, _NP, 16), jnp.float32),
    mesh=_sc_mesh(),
    compiler_params=pltpu.CompilerParams(use_tc_tiling_on_sc=False),
    scratch_types=[
        pltpu.VMEM((_CPT, _CH), jnp.int32),
        pltpu.VMEM((_CH, 16), jnp.float32),
        pltpu.VMEM((_RPS, 16), jnp.float32),
        pltpu.VMEM_SHARED((_NP, 16), jnp.float32),
    ],
)

_BN = 2048


def _tc_layer1_body(p2, d0, d1, W, b, h2, scale, maskb):
  deg = (d0[...] + d1[...])[:, 0:1]
  sc = 1.0 / jnp.maximum(deg, 1.0)
  mk = (deg > 0.0).astype(jnp.float32)
  mean = jnp.concatenate([p2[0], p2[1]], axis=1) * sc
  z = lax.dot_general(mean, W[...], (((1,), (1,)), ((), ())),
                      preferred_element_type=jnp.float32)
  z = z + b[...] * mk
  ht = jnp.tanh(jnp.maximum(z, 0.0))
  h2[0] = ht[:, :_DH]
  h2[1] = ht[:, _DH:]
  scale[...] = sc
  maskb[...] = mk


def _tc_layerk_body(p2, sc_ref, mk_ref, W, b, h2):
  mean = jnp.concatenate([p2[0], p2[1]], axis=1) * sc_ref[...]
  z = lax.dot_general(mean, W[...], (((1,), (1,)), ((), ())),
                      preferred_element_type=jnp.float32)
  z = z + b[...] * mk_ref[...]
  ht = jnp.tanh(jnp.maximum(z, 0.0))
  h2[0] = ht[:, :_DH]
  h2[1] = ht[:, _DH:]


def _blk2():
  return pl.BlockSpec((2, _BN, _DH), lambda i: (0, i, 0))


def _blk(r, c):
  return pl.BlockSpec((r, c), lambda i: (i, 0))


def _fix(r, c):
  return pl.BlockSpec((r, c), lambda i: (0, 0))


def _tc_layer1(p2, d0, d1, W, b):
  return pl.pallas_call(
      _tc_layer1_body,
      grid=(_NP // _BN,),
      in_specs=[_blk2(), _blk(_BN, 16), _blk(_BN, 16),
                _fix(_D, _D), _fix(1, _D)],
      out_specs=[_blk2(), _blk(_BN, 1), _blk(_BN, 1)],
      out_shape=[jax.ShapeDtypeStruct((2, _NP, _DH), jnp.float32),
                 jax.ShapeDtypeStruct((_NP, 1), jnp.float32),
                 jax.ShapeDtypeStruct((_NP, 1), jnp.float32)],
  )(p2, d0, d1, W, b)


def _tc_layerk(p2, scale, maskb, W, b):
  return pl.pallas_call(
      _tc_layerk_body,
      grid=(_NP // _BN,),
      in_specs=[_blk2(), _blk(_BN, 1), _blk(_BN, 1),
                _fix(_D, _D), _fix(1, _D)],
      out_specs=_blk2(),
      out_shape=jax.ShapeDtypeStruct((2, _NP, _DH), jnp.float32),
  )(p2, scale, maskb, W, b)


def _tc_l4pool_body(p2, sc_ref, mk_ref, W, b, bt, Wfc, bfc, Wr, br,
                    out, accs, cnts):
  i = pl.program_id(0)

  @pl.when(i == 0)
  def _():
    accs[...] = jnp.zeros_like(accs)
    cnts[...] = jnp.zeros_like(cnts)

  mean = jnp.concatenate([p2[0], p2[1]], axis=1) * sc_ref[...]
  z = lax.dot_general(mean, W[...], (((1,), (1,)), ((), ())),
                      preferred_element_type=jnp.float32)
  z = z + b[...] * mk_ref[...]
  h4 = jnp.tanh(jnp.maximum(z, 0.0))

  g = lax.broadcasted_iota(jnp.int32, (1, _G), 1)
  M = (bt[...] == g).astype(jnp.float32)
  accs[...] += lax.dot_general(M, h4, (((0,), (0,)), ((), ())),
                               preferred_element_type=jnp.float32)
  cnts[...] += jnp.sum(M, axis=0)[:, None]

  @pl.when(i == pl.num_programs(0) - 1)
  def _():
    pooled = accs[...] / jnp.maximum(cnts[...], 1.0)
    hfc = lax.dot_general(pooled, Wfc[...], (((1,), (1,)), ((), ())),
                          preferred_element_type=jnp.float32) + bfc[...]
    hfc = jnp.maximum(hfc, 0.0)
    out[...] = lax.dot_general(hfc, Wr[...], (((1,), (1,)), ((), ())),
                               preferred_element_type=jnp.float32) + br[...]


def _tc_l4pool(p2, scale, maskb, W, b, bt, Wfc, bfc, Wr, br):
  return pl.pallas_call(
      _tc_l4pool_body,
      grid=(_NP // _BN,),
      in_specs=[_blk2(), _blk(_BN, 1), _blk(_BN, 1), _fix(_D, _D),
                _fix(1, _D), _blk(_BN, 1), _fix(_D, _D), _fix(1, _D),
                _fix(8, _D), _fix(1, 8)],
      out_specs=_fix(_G, 8),
      out_shape=jax.ShapeDtypeStruct((_G, 8), jnp.float32),
      scratch_shapes=[pltpu.VMEM((_G, _D), jnp.float32),
                      pltpu.VMEM((_G, _D), jnp.float32)],
  )(p2, scale, maskb, W, b, bt, Wfc, bfc, Wr, br)


def kernel(x, edge_index, batch, W1, b1, W2, b2, W3, b3, W4, b4,
           Wfc, bfc, Wr, br):
  src16 = edge_index[0].reshape(_NS, _MC, _KC * _CH)
  dst16 = edge_index[1].reshape(_NS, _MC, _KC * _CH)
  dst32 = edge_index[1].reshape(_NW, _CPT, _CH)

  xp = jnp.concatenate(
      [x, jnp.zeros((_NP - _N, _D), jnp.float32)]).reshape(_NP, 2, _DH)
  x2 = jnp.transpose(xp, (1, 0, 2))

  pdeg = _deg(dst32)
  p2 = _agg(x2, src16, dst16)
  h2, scale, maskb = _tc_layer1(p2, pdeg[0], pdeg[1], W1, b1.reshape(1, _D))
  for W, b in ((W2, b2), (W3, b3)):
    p2 = _agg(h2, src16, dst16)
    h2 = _tc_layerk(p2, scale, maskb, W, b.reshape(1, _D))
  p2 = _agg(h2, src16, dst16)

  batch_p = jnp.concatenate(
      [batch, jnp.full((_NP - _N,), _G, jnp.int32)]).reshape(_NP, 1)
  out = _tc_l4pool(p2, scale, maskb, W4, b4.reshape(1, _D), batch_p,
                   Wfc, bfc.reshape(1, _D), Wr, br.reshape(1, 8))
  return out
